# baseline (speedup 1.0000x reference)
import jax
import jax.numpy as jnp
from jax.experimental import pallas as pl
from jax.experimental.pallas import tpu as pltpu

N_NODES = 10000
B = 16


def _mlp_body(g_ref, Wc1_ref, bc1_ref, Wc2_ref, bc2_ref, Wc3_ref, bc3_ref, o_ref):
    g = g_ref[...]
    o = jnp.maximum(jnp.dot(g, Wc1_ref[...], preferred_element_type=jnp.float32) + bc1_ref[...], 0.0)
    o = jnp.maximum(jnp.dot(o, Wc2_ref[...], preferred_element_type=jnp.float32) + bc2_ref[...], 0.0)
    o_ref[...] = jnp.dot(o, Wc3_ref[...], preferred_element_type=jnp.float32) + bc3_ref[...]


def _gatv2(x, src, dst, Wl, Wr, att, b):
    xl = x @ Wl
    xr = x @ Wr
    e = jax.nn.leaky_relu(xl[src] + xr[dst], 0.2) @ att
    e_max = jax.ops.segment_max(e, dst, num_segments=N_NODES)
    e_max = jnp.where(jnp.isfinite(e_max), e_max, 0.0)
    w = jnp.exp(e - e_max[dst])
    denom = jax.ops.segment_sum(w, dst, num_segments=N_NODES)
    alpha = w / (denom[dst] + 1e-16)
    return jax.ops.segment_sum(alpha[:, None] * xl[src], dst, num_segments=N_NODES) + b


def kernel(x, edge_index, batch, Wl1, Wr1, att1, b1, Wl2, Wr2, att2, b2, Wl3, Wr3, att3, b3, Wl4, Wr4, att4, b4, Wc1, bc1, Wc2, bc2, Wc3, bc3):
    src = edge_index[0]
    dst = edge_index[1]
    h = _gatv2(x, src, dst, Wl1, Wr1, att1, b1)
    h = jax.nn.leaky_relu(h, 0.01)
    h = _gatv2(h, src, dst, Wl2, Wr2, att2, b2)
    h = jax.nn.leaky_relu(h, 0.01)
    h = _gatv2(h, src, dst, Wl3, Wr3, att3, b3)
    h = jax.nn.leaky_relu(h, 0.01)
    h = _gatv2(h, src, dst, Wl4, Wr4, att4, b4)
    g = jax.ops.segment_max(h, batch, num_segments=B)
    g = jnp.where(jnp.isfinite(g), g, 0.0)
    out = pl.pallas_call(
        _mlp_body,
        out_shape=jax.ShapeDtypeStruct((B, 4), jnp.float32),
    )(g, Wc1, bc1, Wc2, bc2, Wc3, bc3)
    return out


# speedup vs baseline: 2.3171x; 2.3171x over previous
import functools

import jax
import jax.numpy as jnp
from jax import lax
from jax.experimental import pallas as pl
from jax.experimental.pallas import tpu as pltpu
from jax.experimental.pallas import tpu_sc as plsc

N_NODES = 10000
N_EDGES = 320000
NB = 16

NC, NS, L16 = 2, 16, 16
NW = NC * NS
N_PAD = 10240
DUMMY = N_NODES
EPT = 10240
BLK = 128
NBLK = EPT // BLK
W = 128
ROWS_PER_TILE = N_PAD // NS

_mesh = plsc.VectorSubcoreMesh(core_axis_name="c", subcore_axis_name="s")
_CP_SC = pltpu.CompilerParams(needs_layout_passes=False)


def _mm_body(h_ref, w_ref, o_ref):
    ci = pl.program_id(2)

    @pl.when(ci == 0)
    def _():
        o_ref[...] = jnp.zeros_like(o_ref)

    o_ref[0, ...] += jnp.dot(h_ref[0, ...], w_ref[0, 0, ...],
                             preferred_element_type=jnp.float32)


def _mm(h3, wcat, c_in, g_out):
    rb = 1024
    grid = (g_out, N_PAD // rb, c_in)
    return pl.pallas_call(
        _mm_body,
        grid=grid,
        in_specs=[
            pl.BlockSpec((1, rb, 128), lambda g, r, ci: (ci, r, 0)),
            pl.BlockSpec((1, 1, 128, 128), lambda g, r, ci: (ci, g, 0, 0)),
        ],
        out_specs=pl.BlockSpec((1, rb, 128), lambda g, r, ci: (g, r, 0)),
        out_shape=jax.ShapeDtypeStruct((g_out, N_PAD, 128), jnp.float32),
    )(h3, wcat)


def _mu_body(cnt_ref, sum_ref, mu_ref):
    cnt = jnp.sum(cnt_ref[...], axis=0, keepdims=True)
    tot = jnp.sum(sum_ref[...], axis=0, keepdims=True)
    mu_ref[...] = tot / jnp.maximum(cnt, 1.0)


def _mu_combine(cnt, ssum):
    cb = 2048
    return pl.pallas_call(
        _mu_body,
        grid=(N_PAD // cb,),
        in_specs=[
            pl.BlockSpec((NW, cb), lambda i: (0, i)),
            pl.BlockSpec((NW, cb), lambda i: (0, i)),
        ],
        out_specs=pl.BlockSpec((1, cb), lambda i: (0, i)),
        out_shape=jax.ShapeDtypeStruct((1, N_PAD), jnp.float32),
    )(cnt, ssum)


def _srecip_body(s_ref, o_ref):
    s = jnp.sum(s_ref[...], axis=0, keepdims=True)
    o_ref[...] = 1.0 / (s + 1e-16)


def _srecip_combine(s):
    cb = 2048
    return pl.pallas_call(
        _srecip_body,
        grid=(N_PAD // cb,),
        in_specs=[pl.BlockSpec((NW, cb), lambda i: (0, i))],
        out_specs=pl.BlockSpec((1, cb), lambda i: (0, i)),
        out_shape=jax.ShapeDtypeStruct((1, N_PAD), jnp.float32),
    )(s)


def _hcomb_body(p_ref, b_ref, o_ref):
    lo = p_ref[0, 0] + p_ref[1, 0]
    hi = p_ref[0, 1] + p_ref[1, 1]
    h = jnp.concatenate([lo, hi], axis=-1) + b_ref[0, 0]
    o_ref[0, ...] = jnp.maximum(h, 0.01 * h)


def _h_combine(part64, bp, c_out):
    rb = 1024
    return pl.pallas_call(
        _hcomb_body,
        grid=(c_out, N_PAD // rb),
        in_specs=[
            pl.BlockSpec((NC, 2, rb, 64), lambda cc, r: (0, cc, r, 0)),
            pl.BlockSpec((1, 1, 128), lambda cc, r: (cc, 0, 0)),
        ],
        out_specs=pl.BlockSpec((1, rb, 128), lambda cc, r: (cc, r, 0)),
        out_shape=jax.ShapeDtypeStruct((c_out, N_PAD, 128), jnp.float32),
    )(part64, bp.reshape(c_out, 1, 128))


def _resplit_body(x_ref, o_ref):
    h = pl.program_id(2)

    @pl.when(h == 0)
    def _():
        o_ref[0, ...] = x_ref[0, :, :64]

    @pl.when(h == 1)
    def _():
        o_ref[0, ...] = x_ref[0, :, 64:]


def _resplit(xlxr, c_out):
    rb = 2048
    return pl.pallas_call(
        _resplit_body,
        grid=(c_out, N_PAD // rb, 2),
        in_specs=[pl.BlockSpec((1, rb, 128), lambda c, r, h: (c, r, 0))],
        out_specs=pl.BlockSpec((1, rb, 64), lambda c, r, h: (2 * c + h, r, 0)),
        out_shape=jax.ShapeDtypeStruct((2 * c_out, N_PAD, 64), jnp.float32),
    )(xlxr)


def _make_sc_stats(c_out):
    @functools.partial(
        pl.kernel,
        out_type=(
            jax.ShapeDtypeStruct((NW, NBLK, BLK), jnp.float32),
            jax.ShapeDtypeStruct((NW, N_PAD), jnp.float32),
            jax.ShapeDtypeStruct((NW, N_PAD), jnp.float32),
        ),
        mesh=_mesh,
        scratch_types=[
            pltpu.VMEM((NBLK, BLK), jnp.int32),
            pltpu.VMEM((NBLK, BLK), jnp.int32),
            pltpu.VMEM((NBLK, BLK), jnp.int32),
            pltpu.VMEM((NBLK, BLK), jnp.float32),
            pltpu.VMEM((N_PAD,), jnp.float32),
            pltpu.VMEM((N_PAD,), jnp.float32),
            pltpu.VMEM((BLK, W), jnp.float32),
            pltpu.VMEM((BLK, W), jnp.float32),
            pltpu.VMEM((4, 128), jnp.float32),
            pltpu.SemaphoreType.DMA,
            pltpu.SemaphoreType.DMA,
        ],
        compiler_params=_CP_SC,
    )
    def sc_stats(xlxr, sdp, attp, e_out, cnt_out, sum_out,
                 sdv, srcv, dstv, e_v, cnt_v, sum_v, bufl, bufr, attv, sem, sem2):
        c = lax.axis_index("c")
        s = lax.axis_index("s")
        wid = s * NC + c
        lanes = lax.iota(jnp.int32, 16)
        zero16 = jnp.zeros((16,), jnp.float32)
        ones16 = zero16 + 1.0

        pltpu.sync_copy(sdp.at[wid], sdv)
        pltpu.sync_copy(attp, attv.at[pl.ds(0, c_out)])

        def unpack(j, _):
            def g_unpack(g, _):
                sd = sdv[j, pl.ds(g * 16, 16)]
                srcv[j, pl.ds(g * 16, 16)] = sd & 16383
                dstv[j, pl.ds(g * 16, 16)] = lax.shift_right_logical(sd, 14)
                return 0

            lax.fori_loop(0, BLK // 16, g_unpack, 0)
            return 0

        lax.fori_loop(0, NBLK, unpack, 0)

        def zbody(i, _):
            cnt_v[pl.ds(i * 16, 16)] = zero16
            sum_v[pl.ds(i * 16, 16)] = zero16
            return 0

        lax.fori_loop(0, N_PAD // 16, zbody, 0)

        def blk_body(j, _):
            def g_zero(g, _):
                e_v[j, pl.ds(g * 16, 16)] = zero16
                return 0

            lax.fori_loop(0, BLK // 16, g_zero, 0)

            for cc in range(c_out):
                cpl = pltpu.async_copy(xlxr.at[cc].at[srcv.at[j]], bufl, sem)
                cpr = pltpu.async_copy(xlxr.at[c_out + cc].at[dstv.at[j]], bufr, sem2)
                cpl.wait()
                cpr.wait()

                def g_body(g, _):
                    acc = e_v[j, pl.ds(g * 16, 16)]
                    for e in range(16):
                        dot = jnp.zeros((16,), jnp.float32)
                        for w8 in range(W // 16):
                            x = (bufl[g * 16 + e, pl.ds(w8 * 16, 16)]
                                 + bufr[g * 16 + e, pl.ds(w8 * 16, 16)])
                            x = jnp.maximum(x, 0.2 * x)
                            dot = dot + x * attv[cc, pl.ds(w8 * 16, 16)]
                        val = jnp.sum(dot)
                        acc = jnp.where(lanes == e, acc + val, acc)
                    e_v[j, pl.ds(g * 16, 16)] = acc
                    return 0

                lax.fori_loop(0, BLK // 16, g_body, 0)

            def g_stats(g, _):
                d16 = dstv[j, pl.ds(g * 16, 16)]
                ev16 = e_v[j, pl.ds(g * 16, 16)]
                plsc.addupdate_scatter(cnt_v, [d16], ones16)
                plsc.addupdate_scatter(sum_v, [d16], ev16)
                return 0

            lax.fori_loop(0, BLK // 16, g_stats, 0)
            return 0

        lax.fori_loop(0, NBLK, blk_body, 0)

        pltpu.sync_copy(e_v, e_out.at[wid])
        pltpu.sync_copy(cnt_v, cnt_out.at[wid])
        pltpu.sync_copy(sum_v, sum_out.at[wid])

    return sc_stats


@functools.partial(
    pl.kernel,
    out_type=(
        jax.ShapeDtypeStruct((NW, NBLK, BLK), jnp.float32),
        jax.ShapeDtypeStruct((NW, N_PAD), jnp.float32),
    ),
    mesh=_mesh,
    scratch_types=[
        pltpu.VMEM((NBLK, BLK), jnp.int32),
        pltpu.VMEM((NBLK, BLK), jnp.int32),
        pltpu.VMEM((NBLK, BLK), jnp.float32),
        pltpu.VMEM((N_PAD,), jnp.float32),
        pltpu.VMEM((N_PAD,), jnp.float32),
    ],
    compiler_params=_CP_SC,
)
def _sc_wsum(e_in, sdp, mu, w_out, s_out, sdv, dstv, w_v, mu_v, s_v):
    c = lax.axis_index("c")
    s = lax.axis_index("s")
    wid = s * NC + c
    zero16 = jnp.zeros((16,), jnp.float32)

    pltpu.sync_copy(sdp.at[wid], sdv)
    pltpu.sync_copy(e_in.at[wid], w_v)
    pltpu.sync_copy(mu.at[0], mu_v)

    def unpack(j, _):
        def g_unpack(g, _):
            dstv[j, pl.ds(g * 16, 16)] = lax.shift_right_logical(
                sdv[j, pl.ds(g * 16, 16)], 14)
            return 0

        lax.fori_loop(0, BLK // 16, g_unpack, 0)
        return 0

    lax.fori_loop(0, NBLK, unpack, 0)

    def zbody(i, _):
        s_v[pl.ds(i * 16, 16)] = zero16
        return 0

    lax.fori_loop(0, N_PAD // 16, zbody, 0)

    def body(j, _):
        def g_body(g, _):
            d16 = dstv[j, pl.ds(g * 16, 16)]
            mu16 = plsc.load_gather(mu_v, [d16])
            w16 = jnp.exp(w_v[j, pl.ds(g * 16, 16)] - mu16)
            w_v[j, pl.ds(g * 16, 16)] = w16
            plsc.addupdate_scatter(s_v, [d16], w16)
            return 0

        lax.fori_loop(0, BLK // 16, g_body, 0)
        return 0

    lax.fori_loop(0, NBLK, body, 0)

    pltpu.sync_copy(w_v, w_out.at[wid])
    pltpu.sync_copy(s_v, s_out.at[wid])


def _make_sc_agg(c_out):
    @functools.partial(
        pl.kernel,
        out_type=jax.ShapeDtypeStruct((NC * 2 * c_out, N_PAD, 64), jnp.float32),
        mesh=_mesh,
        scratch_types=[
            pltpu.VMEM((NBLK, BLK), jnp.int32),
            pltpu.VMEM((NBLK, BLK), jnp.int32),
            pltpu.VMEM((NBLK, BLK), jnp.int32),
            pltpu.VMEM((NBLK, BLK), jnp.float32),
            pltpu.VMEM((N_PAD,), jnp.float32),
            pltpu.VMEM((BLK, 64), jnp.float32),
            pltpu.VMEM((128, 64), jnp.float32),
            pltpu.VMEM_SHARED((N_PAD, 64), jnp.float32),
            pltpu.SemaphoreType.DMA,
        ],
        compiler_params=pltpu.CompilerParams(needs_layout_passes=False,
                                             use_tc_tiling_on_sc=False),
    )
    def sc_agg(w_in, sdp, srecip, xl64, part,
               sdv, srcv, dstv, alpha_v, sr_v, buf, zb, shared, sem):
        c = lax.axis_index("c")
        s = lax.axis_index("s")
        wid = s * NC + c
        zero16 = jnp.zeros((16,), jnp.float32)

        pltpu.sync_copy(sdp.at[wid], sdv)
        pltpu.sync_copy(w_in.at[wid], alpha_v)
        pltpu.sync_copy(srecip.at[0], sr_v)

        def unpack(j, _):
            def g_unpack(g, _):
                sd = sdv[j, pl.ds(g * 16, 16)]
                srcv[j, pl.ds(g * 16, 16)] = sd & 16383
                dstv[j, pl.ds(g * 16, 16)] = lax.shift_right_logical(sd, 14)
                return 0

            lax.fori_loop(0, BLK // 16, g_unpack, 0)
            return 0

        lax.fori_loop(0, NBLK, unpack, 0)

        def zb_body(i, _):
            for w8 in range(64 // 16):
                zb[i, pl.ds(w8 * 16, 16)] = zero16
            return 0

        lax.fori_loop(0, 128, zb_body, 0)

        def a_body(j, _):
            def g_body(g, _):
                d16 = dstv[j, pl.ds(g * 16, 16)]
                sr16 = plsc.load_gather(sr_v, [d16])
                alpha_v[j, pl.ds(g * 16, 16)] = alpha_v[j, pl.ds(g * 16, 16)] * sr16
                return 0

            lax.fori_loop(0, BLK // 16, g_body, 0)
            return 0

        lax.fori_loop(0, NBLK, a_body, 0)

        row0 = s * ROWS_PER_TILE
        for cc in range(2 * c_out):
            for k in range(ROWS_PER_TILE // 128):
                pltpu.sync_copy(zb, shared.at[pl.ds(row0 + k * 128, 128)])
            plsc.subcore_barrier()

            def blk_body(j, _):
                pltpu.async_copy(xl64.at[cc].at[srcv.at[j]], buf, sem).wait()

                def g_body(g, _):
                    a16 = alpha_v[j, pl.ds(g * 16, 16)]
                    for e in range(16):
                        ae = jnp.full((16,), a16[e], jnp.float32)
                        for w8 in range(64 // 16):
                            buf[g * 16 + e, pl.ds(w8 * 16, 16)] = (
                                buf[g * 16 + e, pl.ds(w8 * 16, 16)] * ae)
                    return 0

                lax.fori_loop(0, BLK // 16, g_body, 0)
                pltpu.sync_copy(buf, shared.at[dstv.at[j]], add=True)
                return 0

            lax.fori_loop(0, NBLK, blk_body, 0)
            plsc.subcore_barrier()
            pltpu.sync_copy(shared.at[pl.ds(row0, ROWS_PER_TILE)],
                            part.at[c * 2 * c_out + cc].at[pl.ds(row0, ROWS_PER_TILE)])
            plsc.subcore_barrier()

    return sc_agg


def _pool_body(p_ref, b_ref, batch_ref, o_ref):
    @pl.when(pl.program_id(1) == 0)
    def _():
        o_ref[...] = jnp.full_like(o_ref, -jnp.inf)

    h = p_ref[0, 0] + p_ref[1, 0] + b_ref[0, 0]
    batch = batch_ref[...]
    neg = jnp.float32(-jnp.inf)
    for b in range(NB):
        m = jnp.max(jnp.where(batch == b, h, neg), axis=0, keepdims=True)
        o_ref[:, b, :] = jnp.maximum(o_ref[:, b, :], m)


def _pool(part64, bp, batch_pad, c_out):
    rb = 2048
    out = pl.pallas_call(
        _pool_body,
        grid=(2 * c_out, N_PAD // rb),
        in_specs=[
            pl.BlockSpec((NC, 1, rb, 64), lambda cc, r: (0, cc, r, 0)),
            pl.BlockSpec((1, 1, 64), lambda cc, r: (cc, 0, 0)),
            pl.BlockSpec((rb, 1), lambda cc, r: (r, 0)),
        ],
        out_specs=pl.BlockSpec((1, NB, 64), lambda cc, r: (cc, 0, 0)),
        out_shape=jax.ShapeDtypeStruct((2 * c_out, NB, 64), jnp.float32),
    )(part64, bp.reshape(2 * c_out, 1, 64), batch_pad.reshape(N_PAD, 1))
    return out.transpose(1, 0, 2).reshape(NB, c_out * 128)


def _mlp_body(g_ref, w1_ref, b1_ref, w2_ref, b2_ref, w3_ref, b3_ref, o_ref):
    g = g_ref[...]
    g = jnp.where(jnp.isfinite(g), g, 0.0)
    o = jnp.maximum(jnp.dot(g, w1_ref[...], preferred_element_type=jnp.float32)
                    + b1_ref[...], 0.0)
    o = jnp.maximum(jnp.dot(o, w2_ref[...], preferred_element_type=jnp.float32)
                    + b2_ref[...], 0.0)
    o_ref[...] = (jnp.dot(o, w3_ref[...], preferred_element_type=jnp.float32)
                  + b3_ref[...])


def _mlp(g, Wc1, bc1, Wc2, bc2, Wc3, bc3):
    return pl.pallas_call(
        _mlp_body,
        out_shape=jax.ShapeDtypeStruct((NB, 4), jnp.float32),
    )(g, Wc1, bc1, Wc2, bc2, Wc3, bc3)


_LAYER_DIMS = [(128, 64), (64, 128), (128, 256), (256, 512)]


def _pad_to(x, n, axis):
    pad = n - x.shape[axis]
    if pad <= 0:
        return x
    widths = [(0, 0)] * x.ndim
    widths[axis] = (0, pad)
    return jnp.pad(x, widths)


def kernel(x, edge_index, batch, Wl1, Wr1, att1, b1, Wl2, Wr2, att2, b2,
           Wl3, Wr3, att3, b3, Wl4, Wr4, att4, b4, Wc1, bc1, Wc2, bc2, Wc3, bc3):
    Wls = [Wl1, Wl2, Wl3, Wl4]
    Wrs = [Wr1, Wr2, Wr3, Wr4]
    atts = [att1, att2, att3, att4]
    bs = [b1, b2, b3, b4]

    src = edge_index[0]
    dst = edge_index[1]
    pad_e = NW * EPT - N_EDGES
    sd = src | (dst << 14)
    sdp = jnp.concatenate(
        [sd, jnp.full((pad_e,), DUMMY | (DUMMY << 14), jnp.int32)])
    sdp = sdp.reshape(NW, NBLK, BLK)
    batch_pad = jnp.concatenate([batch, jnp.full((N_PAD - N_NODES,), NB, jnp.int32)])

    h3 = _pad_to(x, N_PAD, 0).reshape(1, N_PAD, 128)

    part = None
    for li in range(4):
        din, dout = _LAYER_DIMS[li]
        c_in = max(1, din // 128)
        c_out = max(1, dout // 128)
        wl = _pad_to(_pad_to(Wls[li], c_in * 128, 0), c_out * 128, 1)
        wr = _pad_to(_pad_to(Wrs[li], c_in * 128, 0), c_out * 128, 1)
        wcat = jnp.concatenate([wl, wr], axis=1)
        wcat = wcat.reshape(c_in, 128, 2 * c_out, 128).transpose(0, 2, 1, 3)
        attp = _pad_to(atts[li], c_out * 128, 0).reshape(c_out, 128)
        bp = _pad_to(bs[li], c_out * 128, 0)

        xlxr = _mm(h3, wcat, c_in, 2 * c_out)
        e, cnt, ssum = _make_sc_stats(c_out)(xlxr, sdp, attp)
        mu = _mu_combine(cnt, ssum)
        w, s = _sc_wsum(e, sdp, mu)
        srecip = _srecip_combine(s)
        xl64 = _resplit(xlxr, c_out)
        part = _make_sc_agg(c_out)(w, sdp, srecip, xl64)
        part = part.reshape(NC, 2 * c_out, N_PAD, 64)
        if li < 3:
            h3 = _h_combine(part, bp, c_out)

    g = _pool(part, _pad_to(b4, 512, 0), batch_pad, 4)
    out = _mlp(g, Wc1, bc1, Wc2, bc2, Wc3, bc3)
    return out
